# baseline (speedup 1.0000x reference)
import functools

import jax
import jax.numpy as jnp
from jax import lax
from jax.experimental import pallas as pl
from jax.experimental.pallas import tpu as pltpu
from jax.experimental.pallas import tpu_sc as plsc

_NUM_CLASSES = 100000
_FEAT_DIM = 64
_BATCH = 16384

_INFO = plsc.get_sparse_core_info()
_NC = _INFO.num_cores
_NS = _INFO.num_subcores
_LANES = _INFO.num_lanes
_NW = _NC * _NS
_BPW = _BATCH // _NW
_CHUNK = 128
_NCHUNK = _BPW // _CHUNK


def _sc_body(features_hbm, labels_hbm, centers_hbm, out_hbm,
             idx_v, rows_v, feats_v, acc_v, gsem, fsem):
    wid = lax.axis_index("s") * _NC + lax.axis_index("c")
    base = wid * _BPW

    lcopies = [
        pltpu.async_copy(labels_hbm.at[pl.ds(base + j * _CHUNK, _CHUNK)],
                         idx_v.at[j], gsem.at[j])
        for j in range(_NCHUNK)
    ]
    fcopy = pltpu.async_copy(features_hbm.at[pl.ds(base, _BPW)],
                             feats_v, fsem)

    gathers = []
    for j in range(_NCHUNK):
        lcopies[j].wait()
        gathers.append(
            pltpu.async_copy(centers_hbm.at[idx_v.at[j]],
                             rows_v.at[pl.ds(j * _CHUNK, _CHUNK)],
                             gsem.at[j]))
    fcopy.wait()

    zero = jnp.zeros((_LANES,), jnp.float32)
    nacc = _FEAT_DIM // _LANES

    def body(i, accs):
        out = []
        for k in range(nacc):
            f = feats_v[i, pl.ds(k * _LANES, _LANES)]
            c = rows_v[i, pl.ds(k * _LANES, _LANES)]
            d = f - c
            out.append(accs[k] + d * d)
        return tuple(out)

    accs = (zero,) * nacc
    for j in range(_NCHUNK):
        gathers[j].wait()
        accs = lax.fori_loop(j * _CHUNK, (j + 1) * _CHUNK, body, accs)
    total = (accs[0] + accs[1]) + (accs[2] + accs[3])
    acc_v[...] = total * jnp.float32(1.0 / (_BATCH * _FEAT_DIM))
    pltpu.sync_copy(acc_v, out_hbm.at[wid])


@functools.partial(jax.jit, static_argnames=())
def kernel(features, labels, centers):
    mesh = plsc.VectorSubcoreMesh(core_axis_name="c", subcore_axis_name="s")
    partials = pl.kernel(
        _sc_body,
        mesh=mesh,
        out_type=jax.ShapeDtypeStruct((_NW, _LANES), jnp.float32),
        scratch_types=[
            pltpu.VMEM((_NCHUNK, _CHUNK), jnp.int32),
            pltpu.VMEM((_BPW, _FEAT_DIM), jnp.float32),
            pltpu.VMEM((_BPW, _FEAT_DIM), jnp.float32),
            pltpu.VMEM((_LANES,), jnp.float32),
            pltpu.SemaphoreType.DMA((_NCHUNK,)),
            pltpu.SemaphoreType.DMA,
        ],
        compiler_params=pltpu.CompilerParams(use_tc_tiling_on_sc=False),
    )(features, labels.astype(jnp.int32), centers)
    return jnp.sum(partials)


# speedup vs baseline: 1.3048x; 1.3048x over previous
import functools

import jax
import jax.numpy as jnp
from jax import lax
from jax.experimental import pallas as pl
from jax.experimental.pallas import tpu as pltpu
from jax.experimental.pallas import tpu_sc as plsc

_NUM_CLASSES = 100000
_FEAT_DIM = 64
_BATCH = 16384

_INFO = plsc.get_sparse_core_info()
_NC = _INFO.num_cores
_NS = _INFO.num_subcores
_LANES = _INFO.num_lanes
_NW = _NC * _NS
_BPW = _BATCH // _NW
_CHUNK = 128
_NCHUNK = _BPW // _CHUNK
_GRP = _CHUNK // _LANES


def _sc_body(features_hbm, labels_hbm, centers_hbm, out_hbm,
             lbl_v, feats_v, rows_a, rows_b, acc_v, gsem, fsem, lsem):
    wid = lax.axis_index("s") * _NC + lax.axis_index("c")
    base = wid * _BPW

    pltpu.async_copy(labels_hbm.at[pl.ds(base, _BPW)], lbl_v, lsem).wait()
    fcopy = pltpu.async_copy(features_hbm.at[pl.ds(base, _BPW)],
                             feats_v, fsem)
    bufs = (rows_a, rows_b)

    def issue_chunk(c, buf):
        def grp(g, _):
            lvec = lbl_v[pl.ds(c * _CHUNK + g * _LANES, _LANES)]
            for j in range(_LANES):
                pltpu.async_copy(centers_hbm.at[lvec[j]],
                                 buf.at[g * _LANES + j], gsem.at[c % 2])
            return 0

        lax.fori_loop(0, _GRP, grp, 0)

    def drain_chunk(c, buf):
        def grp(g, _):
            for j in range(_LANES):
                pltpu.make_async_copy(
                    centers_hbm.at[0], buf.at[g * _LANES + j],
                    gsem.at[c % 2]).wait()
            return 0

        lax.fori_loop(0, _GRP, grp, 0)

    zero = jnp.zeros((_LANES,), jnp.float32)
    nacc = _FEAT_DIM // _LANES

    def compute_chunk(c, buf, accs):
        def body(i, accs):
            out = []
            for k in range(nacc):
                f = feats_v[c * _CHUNK + i, pl.ds(k * _LANES, _LANES)]
                cc = buf[i, pl.ds(k * _LANES, _LANES)]
                d = f - cc
                out.append(accs[k] + d * d)
            return tuple(out)

        return lax.fori_loop(0, _CHUNK, body, accs)

    issue_chunk(0, bufs[0])
    accs = (zero,) * nacc
    fwaited = False
    for c in range(_NCHUNK):
        if c + 1 < _NCHUNK:
            issue_chunk(c + 1, bufs[(c + 1) % 2])
        drain_chunk(c, bufs[c % 2])
        if not fwaited:
            fcopy.wait()
            fwaited = True
        accs = compute_chunk(c, bufs[c % 2], accs)

    total = (accs[0] + accs[1]) + (accs[2] + accs[3])
    acc_v[...] = total * jnp.float32(1.0 / (_BATCH * _FEAT_DIM))
    pltpu.sync_copy(acc_v, out_hbm.at[wid])


@functools.partial(jax.jit, static_argnames=())
def kernel(features, labels, centers):
    mesh = plsc.VectorSubcoreMesh(core_axis_name="c", subcore_axis_name="s")
    partials = pl.kernel(
        _sc_body,
        mesh=mesh,
        out_type=jax.ShapeDtypeStruct((_NW, _LANES), jnp.float32),
        scratch_types=[
            pltpu.VMEM((_BPW,), jnp.int32),
            pltpu.VMEM((_BPW, _FEAT_DIM), jnp.float32),
            pltpu.VMEM((_CHUNK, _FEAT_DIM), jnp.float32),
            pltpu.VMEM((_CHUNK, _FEAT_DIM), jnp.float32),
            pltpu.VMEM((_LANES,), jnp.float32),
            pltpu.SemaphoreType.DMA((2,)),
            pltpu.SemaphoreType.DMA,
            pltpu.SemaphoreType.DMA,
        ],
        compiler_params=pltpu.CompilerParams(use_tc_tiling_on_sc=True),
    )(features, labels.astype(jnp.int32), centers)
    return jnp.sum(partials)
